# baseline (speedup 1.0000x reference)
import jax
import jax.numpy as jnp
from jax import lax
from jax.experimental import pallas as pl
from jax.experimental.pallas import tpu as pltpu
from jax.experimental.pallas import tpu_sc as plsc

N = 10000
E = 320000
D = 128
C = 64

NC = 2
NS = 16
NW = NC * NS
K = 80
EPT = E // NW
NCHUNK = EPT // K
ZR = 80
NZCH = N // ZR
ZITER = (NZCH + NS - 1) // NS
LANES = 16


def _zero_rows(ref, nrows):
  zv = jnp.zeros((LANES,), jnp.float32)

  def zr(i, c):
    for j in range(D // LANES):
      ref[i, pl.ds(j * LANES, LANES)] = zv
    return c
  lax.fori_loop(0, nrows, zr, 0)


def _zero_shared(sid, zsrc, acc_sh):
  def zs(t, carry):
    ch = t * NS + sid

    @pl.when(ch < NZCH)
    def _():
      pltpu.sync_copy(zsrc, acc_sh.at[pl.ds(ch * ZR, ZR), :])
    return carry
  lax.fori_loop(0, ZITER, zs, 0)


def _writeback(cid, sid, acc_sh, bounce_v, acc_out):
  def wb(t, carry):
    ch = t * NS + sid

    @pl.when(ch < NZCH)
    def _():
      r = ch * ZR
      pltpu.sync_copy(acc_sh.at[pl.ds(r, ZR), :], bounce_v)
      pltpu.sync_copy(bounce_v, acc_out.at[cid, pl.ds(r, ZR), :])
    return carry
  lax.fori_loop(0, ZITER, wb, 0)


QITER = (NCHUNK + 3) // 4 + 1


def _sc_layer_body(feat, src, dst, xe, acc_out, src_v0, src_v1, src_v2,
                   src_v3, dst_v0, dst_v1, dst_v2, dst_v3, rows0, rows1,
                   xe0, xe1, acc_sh, sem_i0, sem_i1, sem_g0, sem_g1,
                   sem_x0, sem_x1, sem_s0, sem_s1):
  cid = lax.axis_index("c")
  sid = lax.axis_index("s")
  wid = cid * NS + sid
  src_v = (src_v0, src_v1, src_v2, src_v3)
  dst_v = (dst_v0, dst_v1, dst_v2, dst_v3)
  rows_v = (rows0, rows1)
  xe_v = (xe0, xe1)
  sem_i = (sem_i0, sem_i1)
  sem_g = (sem_g0, sem_g1)
  sem_x = (sem_x0, sem_x1)
  sem_s = (sem_s0, sem_s1)

  _zero_rows(rows0, ZR)
  _zero_shared(sid, rows0, acc_sh)
  plsc.subcore_barrier()

  ebase = wid * EPT

  def iss_idx(c, b4):
    @pl.when(c < NCHUNK)
    def _():
      base = ebase + c * K
      pltpu.async_copy(src.at[pl.ds(base, K)], src_v[b4], sem_i[b4 % 2])
      pltpu.async_copy(dst.at[pl.ds(base, K)], dst_v[b4], sem_i[b4 % 2])

  def wait_idx(c, b4):
    @pl.when(c < NCHUNK)
    def _():
      base = ebase + c * K
      pltpu.make_async_copy(src.at[pl.ds(base, K)], src_v[b4],
                            sem_i[b4 % 2]).wait()
      pltpu.make_async_copy(dst.at[pl.ds(base, K)], dst_v[b4],
                            sem_i[b4 % 2]).wait()

  def iss_gather(c, b4, b2):
    @pl.when(c < NCHUNK)
    def _():
      pltpu.async_copy(feat.at[src_v[b4]], rows_v[b2], sem_g[b2])

  def iss_xe(c, b2):
    @pl.when(c < NCHUNK)
    def _():
      base = ebase + c * K
      pltpu.async_copy(xe.at[pl.ds(base, K), :], xe_v[b2], sem_x[b2])

  iss_idx(0, 0)
  iss_idx(1, 1)
  wait_idx(0, 0)
  iss_gather(0, 0, 0)
  iss_xe(0, 0)

  def quad(q, carry):
    for b4 in (0, 1, 2, 3):
      c = 4 * q + b4
      b2 = b4 % 2
      n2 = 1 - b2
      wait_idx(c + 1, (b4 + 1) % 4)
      iss_gather(c + 1, (b4 + 1) % 4, n2)

      @pl.when(c < NCHUNK)
      def _():
        base = ebase + c * K
        pltpu.make_async_copy(feat.at[src_v[b4]], rows_v[b2],
                              sem_g[b2]).wait()
        pltpu.make_async_copy(xe.at[pl.ds(base, K), :], xe_v[b2],
                              sem_x[b2]).wait()

        @plsc.parallel_loop(0, K, unroll=8)
        def _pl(i):
          for j in range(D // LANES):
            s = pl.ds(j * LANES, LANES)
            xe_v[b2][i, s] = jnp.maximum(rows_v[b2][i, s] + xe_v[b2][i, s],
                                         0.0)

        pltpu.async_copy(xe_v[b2], acc_sh.at[dst_v[b4]], sem_s[b2],
                         add=True)

      @pl.when((c >= 1) & (c <= NCHUNK))
      def _():
        pltpu.make_async_copy(xe_v[n2], acc_sh.at[dst_v[(b4 + 3) % 4]],
                              sem_s[n2]).wait()

      iss_xe(c + 1, n2)
      iss_idx(c + 2, (b4 + 2) % 4)
    return carry
  lax.fori_loop(0, QITER, quad, 0)

  plsc.subcore_barrier()
  _writeback(cid, sid, acc_sh, rows0, acc_out)


CW = 128


def _sc_cnt_body(dst, cnt_out, dst_v0, dst_v1, dst_v2, dst_v3, ones_v,
                 cnt_sh, sem_i0, sem_i1, sem_s0, sem_s1):
  cid = lax.axis_index("c")
  sid = lax.axis_index("s")
  wid = cid * NS + sid
  dst_v = (dst_v0, dst_v1, dst_v2, dst_v3)
  sem_i = (sem_i0, sem_i1)
  sem_s = (sem_s0, sem_s1)

  zv = jnp.zeros((LANES,), jnp.float32)

  def zr(i, c):
    for j in range(CW // LANES):
      ones_v[i, pl.ds(j * LANES, LANES)] = zv
    return c
  lax.fori_loop(0, ZR, zr, 0)
  _zero_shared(sid, ones_v, cnt_sh)

  lane = lax.iota(jnp.int32, LANES)
  one_row = jnp.where(lane == 0, 1.0, 0.0).astype(jnp.float32)

  def init_ones(i, carry):
    ones_v[i, pl.ds(0, LANES)] = one_row
    return carry
  lax.fori_loop(0, K, init_ones, 0)

  plsc.subcore_barrier()

  ebase = wid * EPT

  def iss_idx(c, b4):
    @pl.when(c < NCHUNK)
    def _():
      base = ebase + c * K
      pltpu.async_copy(dst.at[pl.ds(base, K)], dst_v[b4], sem_i[b4 % 2])

  iss_idx(0, 0)
  iss_idx(1, 1)

  def quad(q, carry):
    for b4 in (0, 1, 2, 3):
      c = 4 * q + b4

      @pl.when(c < NCHUNK)
      def _():
        base = ebase + c * K
        pltpu.make_async_copy(dst.at[pl.ds(base, K)], dst_v[b4],
                              sem_i[b4 % 2]).wait()
        pltpu.async_copy(ones_v, cnt_sh.at[dst_v[b4]], sem_s[b4 % 2],
                         add=True)

      @pl.when((c >= 1) & (c <= NCHUNK))
      def _():
        pltpu.make_async_copy(ones_v, cnt_sh.at[dst_v[(b4 + 3) % 4]],
                              sem_s[(b4 + 1) % 2]).wait()

      iss_idx(c + 2, (b4 + 2) % 4)
    return carry
  lax.fori_loop(0, QITER, quad, 0)

  plsc.subcore_barrier()

  def wb(t, carry):
    ch = t * NS + sid

    @pl.when(ch < NZCH)
    def _():
      r = ch * ZR
      pltpu.sync_copy(cnt_sh.at[pl.ds(r, ZR), :], ones_v)
      pltpu.sync_copy(ones_v, cnt_out.at[cid, pl.ds(r, ZR), :])
    return carry
  lax.fori_loop(0, ZITER, wb, 0)


_MESH = plsc.VectorSubcoreMesh(core_axis_name="c", subcore_axis_name="s",
                               num_cores=NC, num_subcores=NS)

_sc_layer = pl.kernel(
    _sc_layer_body,
    out_type=jax.ShapeDtypeStruct((NC, N, D), jnp.float32),
    mesh=_MESH,
    scratch_types=[
        pltpu.VMEM((K,), jnp.int32),
        pltpu.VMEM((K,), jnp.int32),
        pltpu.VMEM((K,), jnp.int32),
        pltpu.VMEM((K,), jnp.int32),
        pltpu.VMEM((K,), jnp.int32),
        pltpu.VMEM((K,), jnp.int32),
        pltpu.VMEM((K,), jnp.int32),
        pltpu.VMEM((K,), jnp.int32),
        pltpu.VMEM((K, D), jnp.float32),
        pltpu.VMEM((K, D), jnp.float32),
        pltpu.VMEM((K, D), jnp.float32),
        pltpu.VMEM((K, D), jnp.float32),
        pltpu.VMEM_SHARED((N, D), jnp.float32),
        pltpu.SemaphoreType.DMA,
        pltpu.SemaphoreType.DMA,
        pltpu.SemaphoreType.DMA,
        pltpu.SemaphoreType.DMA,
        pltpu.SemaphoreType.DMA,
        pltpu.SemaphoreType.DMA,
        pltpu.SemaphoreType.DMA,
        pltpu.SemaphoreType.DMA,
    ],
)

_sc_cnt = pl.kernel(
    _sc_cnt_body,
    out_type=jax.ShapeDtypeStruct((NC, N, CW), jnp.float32),
    mesh=_MESH,
    scratch_types=[
        pltpu.VMEM((K,), jnp.int32),
        pltpu.VMEM((K,), jnp.int32),
        pltpu.VMEM((K,), jnp.int32),
        pltpu.VMEM((K,), jnp.int32),
        pltpu.VMEM((ZR, CW), jnp.float32),
        pltpu.VMEM_SHARED((N, CW), jnp.float32),
        pltpu.SemaphoreType.DMA,
        pltpu.SemaphoreType.DMA,
        pltpu.SemaphoreType.DMA,
        pltpu.SemaphoreType.DMA,
    ],
)

BT = 2000


def _tc1_body(acc_ref, cnt_ref, x_ref, wl_ref, bl_ref, wr_ref, o_ref):
  a = acc_ref[...]
  s = a[0] + a[1]
  cc = cnt_ref[...]
  c = cc[0][:, 0:1] + cc[1][:, 0:1]
  agg = s / jnp.maximum(c, 1.0)
  h = (jnp.dot(agg, wl_ref[...], preferred_element_type=jnp.float32)
       + bl_ref[...]
       + jnp.dot(x_ref[...], wr_ref[...], preferred_element_type=jnp.float32))
  o_ref[...] = jnp.maximum(h, 0.0)


_tc1 = pl.pallas_call(
    _tc1_body,
    grid=(N // BT,),
    in_specs=[
        pl.BlockSpec((NC, BT, D), lambda i: (0, i, 0)),
        pl.BlockSpec((NC, BT, CW), lambda i: (0, i, 0)),
        pl.BlockSpec((BT, D), lambda i: (i, 0)),
        pl.BlockSpec((D, D), lambda i: (0, 0)),
        pl.BlockSpec((1, D), lambda i: (0, 0)),
        pl.BlockSpec((D, D), lambda i: (0, 0)),
    ],
    out_specs=pl.BlockSpec((BT, D), lambda i: (i, 0)),
    out_shape=jax.ShapeDtypeStruct((N, D), jnp.float32),
)


def _tc2_body(acc_ref, cnt_ref, h_ref, wl_ref, bl_ref, wr_ref, wp_ref,
              bp_ref, o_ref, g_ref):
  i = pl.program_id(0)
  a = acc_ref[...]
  s = a[0] + a[1]
  cc = cnt_ref[...]
  c = cc[0][:, 0:1] + cc[1][:, 0:1]
  agg = s / jnp.maximum(c, 1.0)
  h2 = (jnp.dot(agg, wl_ref[...], preferred_element_type=jnp.float32)
        + bl_ref[...]
        + jnp.dot(h_ref[...], wr_ref[...], preferred_element_type=jnp.float32))
  o_ref[...] = (jnp.dot(jnp.maximum(h2, 0.0), wp_ref[...],
                        preferred_element_type=jnp.float32) + bp_ref[...])
  part = jnp.sum(h2, axis=0, keepdims=True) * (1.0 / N)

  @pl.when(i == 0)
  def _():
    g_ref[...] = part

  @pl.when(i != 0)
  def _():
    g_ref[...] = g_ref[...] + part


_tc2 = pl.pallas_call(
    _tc2_body,
    grid=(N // BT,),
    in_specs=[
        pl.BlockSpec((NC, BT, D), lambda i: (0, i, 0)),
        pl.BlockSpec((NC, BT, CW), lambda i: (0, i, 0)),
        pl.BlockSpec((BT, D), lambda i: (i, 0)),
        pl.BlockSpec((D, D), lambda i: (0, 0)),
        pl.BlockSpec((1, D), lambda i: (0, 0)),
        pl.BlockSpec((D, D), lambda i: (0, 0)),
        pl.BlockSpec((D, C), lambda i: (0, 0)),
        pl.BlockSpec((1, C), lambda i: (0, 0)),
    ],
    out_specs=[
        pl.BlockSpec((BT, C), lambda i: (i, 0)),
        pl.BlockSpec((1, D), lambda i: (0, 0)),
    ],
    out_shape=[
        jax.ShapeDtypeStruct((N, C), jnp.float32),
        jax.ShapeDtypeStruct((1, D), jnp.float32),
    ],
)


def kernel(x, edge_index, xe, W_l0, b_l0, W_r0, W_l1, b_l1, W_r1, W_proj,
           b_proj):
  src = edge_index[0]
  dst = edge_index[1]
  cnt = _sc_cnt(dst)
  acc0 = _sc_layer(x, src, dst, xe)
  h = _tc1(acc0, cnt, x, W_l0, b_l0.reshape(1, D), W_r0)
  acc1 = _sc_layer(h, src, dst, xe)
  h_out, g = _tc2(acc1, cnt, h, W_l1, b_l1.reshape(1, D), W_r1, W_proj,
                  b_proj.reshape(1, C))
  return (h_out, g)


# speedup vs baseline: 1.0040x; 1.0040x over previous
import jax
import jax.numpy as jnp
from jax import lax
from jax.experimental import pallas as pl
from jax.experimental.pallas import tpu as pltpu
from jax.experimental.pallas import tpu_sc as plsc

N = 10000
E = 320000
D = 128
C = 64

NC = 2
NS = 16
NW = NC * NS
K = 80
EPT = E // NW
NCHUNK = EPT // K
ZR = 80
NZCH = N // ZR
ZITER = (NZCH + NS - 1) // NS
LANES = 16


def _zero_rows(ref, nrows):
  zv = jnp.zeros((LANES,), jnp.float32)

  def zr(i, c):
    for j in range(D // LANES):
      ref[i, pl.ds(j * LANES, LANES)] = zv
    return c
  lax.fori_loop(0, nrows, zr, 0)


def _zero_shared(sid, zsrc, acc_sh):
  def zs(t, carry):
    ch = t * NS + sid

    @pl.when(ch < NZCH)
    def _():
      pltpu.sync_copy(zsrc, acc_sh.at[pl.ds(ch * ZR, ZR), :])
    return carry
  lax.fori_loop(0, ZITER, zs, 0)


def _writeback(cid, sid, acc_sh, bounce_v, acc_out):
  def wb(t, carry):
    ch = t * NS + sid

    @pl.when(ch < NZCH)
    def _():
      r = ch * ZR
      pltpu.sync_copy(acc_sh.at[pl.ds(r, ZR), :], bounce_v)
      pltpu.sync_copy(bounce_v, acc_out.at[cid, pl.ds(r, ZR), :])
    return carry
  lax.fori_loop(0, ZITER, wb, 0)


QITER = (NCHUNK + 3) // 4 + 1


def _sc_layer_body(feat, src, dst, xe, acc_out, src_v0, src_v1, src_v2,
                   src_v3, dst_v0, dst_v1, dst_v2, dst_v3, rows0, rows1,
                   xe0, xe1, acc_sh, sem_i0, sem_i1, sem_g0, sem_g1,
                   sem_x0, sem_x1, sem_s0, sem_s1):
  cid = lax.axis_index("c")
  sid = lax.axis_index("s")
  wid = cid * NS + sid
  src_v = (src_v0, src_v1, src_v2, src_v3)
  dst_v = (dst_v0, dst_v1, dst_v2, dst_v3)
  rows_v = (rows0, rows1)
  xe_v = (xe0, xe1)
  sem_i = (sem_i0, sem_i1)
  sem_g = (sem_g0, sem_g1)
  sem_x = (sem_x0, sem_x1)
  sem_s = (sem_s0, sem_s1)

  _zero_rows(rows0, ZR)
  _zero_shared(sid, rows0, acc_sh)
  plsc.subcore_barrier()

  ebase = wid * EPT

  def iss_idx(c, b4):
    @pl.when(c < NCHUNK)
    def _():
      base = ebase + c * K
      pltpu.async_copy(src.at[pl.ds(base, K)], src_v[b4], sem_i[b4 % 2])
      pltpu.async_copy(dst.at[pl.ds(base, K)], dst_v[b4], sem_i[b4 % 2])

  def wait_idx(c, b4):
    @pl.when(c < NCHUNK)
    def _():
      base = ebase + c * K
      pltpu.make_async_copy(src.at[pl.ds(base, K)], src_v[b4],
                            sem_i[b4 % 2]).wait()
      pltpu.make_async_copy(dst.at[pl.ds(base, K)], dst_v[b4],
                            sem_i[b4 % 2]).wait()

  def iss_gather(c, b4, b2):
    @pl.when(c < NCHUNK)
    def _():
      pltpu.async_copy(feat.at[src_v[b4]], rows_v[b2], sem_g[b2])

  def iss_xe(c, b2):
    @pl.when(c < NCHUNK)
    def _():
      base = ebase + c * K
      pltpu.async_copy(xe.at[pl.ds(base, K), :], xe_v[b2], sem_x[b2])

  iss_idx(0, 0)
  iss_idx(1, 1)
  wait_idx(0, 0)
  iss_gather(0, 0, 0)
  iss_xe(0, 0)

  def quad(q, carry):
    for b4 in (0, 1, 2, 3):
      c = 4 * q + b4
      b2 = b4 % 2
      n2 = 1 - b2
      wait_idx(c + 1, (b4 + 1) % 4)
      iss_gather(c + 1, (b4 + 1) % 4, n2)

      @pl.when(c < NCHUNK)
      def _():
        base = ebase + c * K
        pltpu.make_async_copy(feat.at[src_v[b4]], rows_v[b2],
                              sem_g[b2]).wait()
        pltpu.make_async_copy(xe.at[pl.ds(base, K), :], xe_v[b2],
                              sem_x[b2]).wait()

        @plsc.parallel_loop(0, K, unroll=4)
        def _pl(i):
          for j in range(D // LANES):
            s = pl.ds(j * LANES, LANES)
            xe_v[b2][i, s] = jnp.maximum(rows_v[b2][i, s] + xe_v[b2][i, s],
                                         0.0)

        pltpu.async_copy(xe_v[b2], acc_sh.at[dst_v[b4]], sem_s[b2],
                         add=True)

      @pl.when((c >= 1) & (c <= NCHUNK))
      def _():
        pltpu.make_async_copy(xe_v[n2], acc_sh.at[dst_v[(b4 + 3) % 4]],
                              sem_s[n2]).wait()

      iss_xe(c + 1, n2)
      iss_idx(c + 2, (b4 + 2) % 4)
    return carry
  lax.fori_loop(0, QITER, quad, 0)

  plsc.subcore_barrier()
  _writeback(cid, sid, acc_sh, rows0, acc_out)


CW = 128


def _sc_cnt_body(dst, cnt_out, dst_v0, dst_v1, dst_v2, dst_v3, ones_v,
                 cnt_sh, sem_i0, sem_i1, sem_s0, sem_s1):
  cid = lax.axis_index("c")
  sid = lax.axis_index("s")
  wid = cid * NS + sid
  dst_v = (dst_v0, dst_v1, dst_v2, dst_v3)
  sem_i = (sem_i0, sem_i1)
  sem_s = (sem_s0, sem_s1)

  zv = jnp.zeros((LANES,), jnp.float32)

  def zr(i, c):
    for j in range(CW // LANES):
      ones_v[i, pl.ds(j * LANES, LANES)] = zv
    return c
  lax.fori_loop(0, ZR, zr, 0)
  _zero_shared(sid, ones_v, cnt_sh)

  lane = lax.iota(jnp.int32, LANES)
  one_row = jnp.where(lane == 0, 1.0, 0.0).astype(jnp.float32)

  def init_ones(i, carry):
    ones_v[i, pl.ds(0, LANES)] = one_row
    return carry
  lax.fori_loop(0, K, init_ones, 0)

  plsc.subcore_barrier()

  ebase = wid * EPT

  def iss_idx(c, b4):
    @pl.when(c < NCHUNK)
    def _():
      base = ebase + c * K
      pltpu.async_copy(dst.at[pl.ds(base, K)], dst_v[b4], sem_i[b4 % 2])

  iss_idx(0, 0)
  iss_idx(1, 1)

  def quad(q, carry):
    for b4 in (0, 1, 2, 3):
      c = 4 * q + b4

      @pl.when(c < NCHUNK)
      def _():
        base = ebase + c * K
        pltpu.make_async_copy(dst.at[pl.ds(base, K)], dst_v[b4],
                              sem_i[b4 % 2]).wait()
        pltpu.async_copy(ones_v, cnt_sh.at[dst_v[b4]], sem_s[b4 % 2],
                         add=True)

      @pl.when((c >= 1) & (c <= NCHUNK))
      def _():
        pltpu.make_async_copy(ones_v, cnt_sh.at[dst_v[(b4 + 3) % 4]],
                              sem_s[(b4 + 1) % 2]).wait()

      iss_idx(c + 2, (b4 + 2) % 4)
    return carry
  lax.fori_loop(0, QITER, quad, 0)

  plsc.subcore_barrier()

  def wb(t, carry):
    ch = t * NS + sid

    @pl.when(ch < NZCH)
    def _():
      r = ch * ZR
      pltpu.sync_copy(cnt_sh.at[pl.ds(r, ZR), :], ones_v)
      pltpu.sync_copy(ones_v, cnt_out.at[cid, pl.ds(r, ZR), :])
    return carry
  lax.fori_loop(0, ZITER, wb, 0)


_MESH = plsc.VectorSubcoreMesh(core_axis_name="c", subcore_axis_name="s",
                               num_cores=NC, num_subcores=NS)

_sc_layer = pl.kernel(
    _sc_layer_body,
    out_type=jax.ShapeDtypeStruct((NC, N, D), jnp.float32),
    mesh=_MESH,
    scratch_types=[
        pltpu.VMEM((K,), jnp.int32),
        pltpu.VMEM((K,), jnp.int32),
        pltpu.VMEM((K,), jnp.int32),
        pltpu.VMEM((K,), jnp.int32),
        pltpu.VMEM((K,), jnp.int32),
        pltpu.VMEM((K,), jnp.int32),
        pltpu.VMEM((K,), jnp.int32),
        pltpu.VMEM((K,), jnp.int32),
        pltpu.VMEM((K, D), jnp.float32),
        pltpu.VMEM((K, D), jnp.float32),
        pltpu.VMEM((K, D), jnp.float32),
        pltpu.VMEM((K, D), jnp.float32),
        pltpu.VMEM_SHARED((N, D), jnp.float32),
        pltpu.SemaphoreType.DMA,
        pltpu.SemaphoreType.DMA,
        pltpu.SemaphoreType.DMA,
        pltpu.SemaphoreType.DMA,
        pltpu.SemaphoreType.DMA,
        pltpu.SemaphoreType.DMA,
        pltpu.SemaphoreType.DMA,
        pltpu.SemaphoreType.DMA,
    ],
)

_sc_cnt = pl.kernel(
    _sc_cnt_body,
    out_type=jax.ShapeDtypeStruct((NC, N, CW), jnp.float32),
    mesh=_MESH,
    scratch_types=[
        pltpu.VMEM((K,), jnp.int32),
        pltpu.VMEM((K,), jnp.int32),
        pltpu.VMEM((K,), jnp.int32),
        pltpu.VMEM((K,), jnp.int32),
        pltpu.VMEM((ZR, CW), jnp.float32),
        pltpu.VMEM_SHARED((N, CW), jnp.float32),
        pltpu.SemaphoreType.DMA,
        pltpu.SemaphoreType.DMA,
        pltpu.SemaphoreType.DMA,
        pltpu.SemaphoreType.DMA,
    ],
)

BT = 2000


def _tc1_body(acc_ref, cnt_ref, x_ref, wl_ref, bl_ref, wr_ref, o_ref):
  a = acc_ref[...]
  s = a[0] + a[1]
  cc = cnt_ref[...]
  c = cc[0][:, 0:1] + cc[1][:, 0:1]
  agg = s / jnp.maximum(c, 1.0)
  h = (jnp.dot(agg, wl_ref[...], preferred_element_type=jnp.float32)
       + bl_ref[...]
       + jnp.dot(x_ref[...], wr_ref[...], preferred_element_type=jnp.float32))
  o_ref[...] = jnp.maximum(h, 0.0)


_tc1 = pl.pallas_call(
    _tc1_body,
    grid=(N // BT,),
    in_specs=[
        pl.BlockSpec((NC, BT, D), lambda i: (0, i, 0)),
        pl.BlockSpec((NC, BT, CW), lambda i: (0, i, 0)),
        pl.BlockSpec((BT, D), lambda i: (i, 0)),
        pl.BlockSpec((D, D), lambda i: (0, 0)),
        pl.BlockSpec((1, D), lambda i: (0, 0)),
        pl.BlockSpec((D, D), lambda i: (0, 0)),
    ],
    out_specs=pl.BlockSpec((BT, D), lambda i: (i, 0)),
    out_shape=jax.ShapeDtypeStruct((N, D), jnp.float32),
)


def _tc2_body(acc_ref, cnt_ref, h_ref, wl_ref, bl_ref, wr_ref, wp_ref,
              bp_ref, o_ref, g_ref):
  i = pl.program_id(0)
  a = acc_ref[...]
  s = a[0] + a[1]
  cc = cnt_ref[...]
  c = cc[0][:, 0:1] + cc[1][:, 0:1]
  agg = s / jnp.maximum(c, 1.0)
  h2 = (jnp.dot(agg, wl_ref[...], preferred_element_type=jnp.float32)
        + bl_ref[...]
        + jnp.dot(h_ref[...], wr_ref[...], preferred_element_type=jnp.float32))
  o_ref[...] = (jnp.dot(jnp.maximum(h2, 0.0), wp_ref[...],
                        preferred_element_type=jnp.float32) + bp_ref[...])
  part = jnp.sum(h2, axis=0, keepdims=True) * (1.0 / N)

  @pl.when(i == 0)
  def _():
    g_ref[...] = part

  @pl.when(i != 0)
  def _():
    g_ref[...] = g_ref[...] + part


_tc2 = pl.pallas_call(
    _tc2_body,
    grid=(N // BT,),
    in_specs=[
        pl.BlockSpec((NC, BT, D), lambda i: (0, i, 0)),
        pl.BlockSpec((NC, BT, CW), lambda i: (0, i, 0)),
        pl.BlockSpec((BT, D), lambda i: (i, 0)),
        pl.BlockSpec((D, D), lambda i: (0, 0)),
        pl.BlockSpec((1, D), lambda i: (0, 0)),
        pl.BlockSpec((D, D), lambda i: (0, 0)),
        pl.BlockSpec((D, C), lambda i: (0, 0)),
        pl.BlockSpec((1, C), lambda i: (0, 0)),
    ],
    out_specs=[
        pl.BlockSpec((BT, C), lambda i: (i, 0)),
        pl.BlockSpec((1, D), lambda i: (0, 0)),
    ],
    out_shape=[
        jax.ShapeDtypeStruct((N, C), jnp.float32),
        jax.ShapeDtypeStruct((1, D), jnp.float32),
    ],
)


def kernel(x, edge_index, xe, W_l0, b_l0, W_r0, W_l1, b_l1, W_r1, W_proj,
           b_proj):
  src = edge_index[0]
  dst = edge_index[1]
  cnt = _sc_cnt(dst)
  acc0 = _sc_layer(x, src, dst, xe)
  h = _tc1(acc0, cnt, x, W_l0, b_l0.reshape(1, D), W_r0)
  acc1 = _sc_layer(h, src, dst, xe)
  h_out, g = _tc2(acc1, cnt, h, W_l1, b_l1.reshape(1, D), W_r1, W_proj,
                  b_proj.reshape(1, C))
  return (h_out, g)
